# Initial kernel scaffold; baseline (speedup 1.0000x reference)
import jax
import jax.numpy as jnp
from jax.experimental import pallas as pl


def kernel(x, edge_index, W, b):
    raise NotImplementedError("write your pallas kernel here")


# speedup vs baseline: 23.7889x; 23.7889x over previous
import functools

import jax
import jax.numpy as jnp
from jax import lax
from jax.experimental import pallas as pl
from jax.experimental.pallas import tpu as pltpu
from jax.experimental.pallas import tpu_sc as plsc

N = 10000
E = 160000
D = 256
DH = 128
NP = 10240
NW = 32
NCHUNK = 40
CH = 128
EP = NW * NCHUNK * CH
RPT = NP // 16

_sc_mesh = plsc.VectorSubcoreMesh(core_axis_name="c", subcore_axis_name="s")


def _deg_body(col3, val3, deg_out, idx2d, val2d, zbuf, deg_sh):
    c = lax.axis_index("c")
    s = lax.axis_index("s")
    wid = s * 2 + c

    def _z(i, _):
        zbuf[pl.ds(i * 16, 16)] = jnp.zeros((16,), jnp.float32)
        return 0

    lax.fori_loop(0, RPT // 16, _z, 0)
    pltpu.sync_copy(zbuf, deg_sh.at[pl.ds(s * RPT, RPT)])
    plsc.subcore_barrier()

    pltpu.sync_copy(col3.at[wid], idx2d)
    pltpu.sync_copy(val3.at[wid], val2d)
    for j in range(NCHUNK):
        pltpu.sync_copy(val2d.at[j], deg_sh.at[idx2d.at[j]], add=True)
    plsc.subcore_barrier()
    pltpu.sync_copy(deg_sh.at[pl.ds(s * RPT, RPT)],
                    deg_out.at[c, pl.ds(s * RPT, RPT)])


_deg_kernel = functools.partial(
    pl.kernel,
    out_type=jax.ShapeDtypeStruct((2, NP), jnp.float32),
    mesh=_sc_mesh,
    scratch_types=[
        pltpu.VMEM((NCHUNK, CH), jnp.int32),
        pltpu.VMEM((NCHUNK, CH), jnp.float32),
        pltpu.VMEM((RPT,), jnp.float32),
        pltpu.VMEM_SHARED((NP,), jnp.float32),
    ],
)(_deg_body)


def _agg_body(row4, col4, ycat, acc_out, dst2d, src2d, buf0, buf1, acc_sh,
              sem0, sem1):
    c = lax.axis_index("c")
    s = lax.axis_index("s")

    def _zr(i, _):
        for k in range(8):
            buf0[i, pl.ds(k * 16, 16)] = jnp.zeros((16,), jnp.float32)
        return 0

    lax.fori_loop(0, CH, _zr, 0)
    for q in range(RPT // CH):
        pltpu.sync_copy(
            buf0, acc_sh.at[pl.ds(s * RPT + q * CH, CH)])
    plsc.subcore_barrier()

    off = c * NP
    sems = (sem0, sem1)
    bufs = (buf0, buf1)
    for p in range(2):
        pltpu.sync_copy(row4.at[s, pl.ds(p * NCHUNK, NCHUNK)], dst2d)
        pltpu.sync_copy(col4.at[s, pl.ds(p * NCHUNK, NCHUNK)], src2d)

        def _adj(j, _):
            for k in range(8):
                v = src2d[j, pl.ds(k * 16, 16)]
                src2d[j, pl.ds(k * 16, 16)] = v + off
            return 0

        lax.fori_loop(0, NCHUNK, _adj, 0)

        pending = pltpu.async_copy(ycat.at[src2d.at[0]], buf0, sems[0])
        for j in range(NCHUNK):
            nxt = None
            if j + 1 < NCHUNK:
                nxt = pltpu.async_copy(ycat.at[src2d.at[j + 1]],
                                       bufs[(j + 1) % 2], sems[(j + 1) % 2])
            pending.wait()
            pltpu.sync_copy(bufs[j % 2], acc_sh.at[dst2d.at[j]], add=True)
            pending = nxt

    plsc.subcore_barrier()
    pltpu.sync_copy(acc_sh.at[pl.ds(s * RPT, RPT)],
                    acc_out.at[c, pl.ds(s * RPT, RPT), :])


_agg_kernel = functools.partial(
    pl.kernel,
    out_type=jax.ShapeDtypeStruct((2, NP, DH), jnp.float32),
    mesh=_sc_mesh,
    scratch_types=[
        pltpu.VMEM((NCHUNK, CH), jnp.int32),
        pltpu.VMEM((NCHUNK, CH), jnp.int32),
        pltpu.VMEM((CH, DH), jnp.float32),
        pltpu.VMEM((CH, DH), jnp.float32),
        pltpu.VMEM_SHARED((NP, DH), jnp.float32),
        pltpu.SemaphoreType.DMA,
        pltpu.SemaphoreType.DMA,
    ],
)(_agg_body)


def _lin_body(x_ref, w_ref, b_ref, deg_ref, y_ref):
    xl = lax.dot_general(x_ref[...], w_ref[...], (((1,), (1,)), ((), ())),
                         preferred_element_type=jnp.float32)
    xl = xl + b_ref[...]
    dinv = lax.rsqrt(deg_ref[0, :] + deg_ref[1, :] + 1.0)[:, None]
    y = xl * dinv
    y_ref[0] = y[:, :DH]
    y_ref[1] = y[:, DH:]


def _lin_call(x_pad, w, b2, deg_part):
    blk = NP // 8
    return pl.pallas_call(
        _lin_body,
        grid=(8,),
        in_specs=[
            pl.BlockSpec((blk, D), lambda i: (i, 0)),
            pl.BlockSpec((D, D), lambda i: (0, 0)),
            pl.BlockSpec((1, D), lambda i: (0, 0)),
            pl.BlockSpec((2, blk), lambda i: (0, i)),
        ],
        out_specs=pl.BlockSpec((2, blk, DH), lambda i: (0, i, 0)),
        out_shape=jax.ShapeDtypeStruct((2, NP, DH), jnp.float32),
    )(x_pad, w, b2, deg_part)


def _out_body(acc_ref, y_ref, deg_ref, o_ref):
    dinv = lax.rsqrt(deg_ref[0, :] + deg_ref[1, :] + 1.0)[:, None]
    h0 = (acc_ref[0] + y_ref[0]) * dinv
    h1 = (acc_ref[1] + y_ref[1]) * dinv
    o_ref[...] = jnp.concatenate([h0, h1], axis=1)


def _out_call(acc, y, deg_part):
    blk = NP // 8
    return pl.pallas_call(
        _out_body,
        grid=(8,),
        in_specs=[
            pl.BlockSpec((2, blk, DH), lambda i: (0, i, 0)),
            pl.BlockSpec((2, blk, DH), lambda i: (0, i, 0)),
            pl.BlockSpec((2, blk), lambda i: (0, i)),
        ],
        out_specs=pl.BlockSpec((blk, D), lambda i: (i, 0)),
        out_shape=jax.ShapeDtypeStruct((NP, D), jnp.float32),
    )(acc, y, deg_part)


def kernel(x, edge_index, W, b):
    x_pad = jnp.pad(x, ((0, NP - N), (0, 0)))
    row = edge_index[0]
    col = edge_index[1]
    pad = EP - E
    k = jnp.arange(pad, dtype=jnp.int32)
    rowp = jnp.concatenate([row, N + (k % (NP - N))])
    colp = jnp.concatenate([col, (k * 97) % N])
    vals = jnp.concatenate(
        [jnp.ones((E,), jnp.float32), jnp.zeros((pad,), jnp.float32)])
    row3 = rowp.reshape(NW, NCHUNK, CH)
    col3 = colp.reshape(NW, NCHUNK, CH)
    val3 = vals.reshape(NW, NCHUNK, CH)
    row4 = rowp.reshape(16, 2 * NCHUNK, CH)
    col4 = colp.reshape(16, 2 * NCHUNK, CH)

    deg_part = _deg_kernel(col3, val3)
    y = _lin_call(x_pad, W, b.reshape(1, D), deg_part)
    ycat = y.reshape(2 * NP, DH)
    acc = _agg_kernel(row4, col4, ycat)
    out_pad = _out_call(acc, y, deg_part)
    return out_pad[:N]


# speedup vs baseline: 25.5440x; 1.0738x over previous
import functools

import jax
import jax.numpy as jnp
from jax import lax
from jax.experimental import pallas as pl
from jax.experimental.pallas import tpu as pltpu
from jax.experimental.pallas import tpu_sc as plsc

N = 10000
E = 160000
D = 256
DH = 128
NP = 10240
CH = 125
NCHUNK = 40
RPT = NP // 16

_sc_mesh = plsc.VectorSubcoreMesh(core_axis_name="c", subcore_axis_name="s")


def _deg_body(col3, deg_out, idx2d, ones, zbuf, deg_sh):
    c = lax.axis_index("c")
    s = lax.axis_index("s")
    wid = s * 2 + c

    for k in range(8):
        ones[pl.ds(k * 16, 16)] = jnp.ones((16,), jnp.float32)

    def _z(i, _):
        zbuf[pl.ds(i * 16, 16)] = jnp.zeros((16,), jnp.float32)
        return 0

    lax.fori_loop(0, RPT // 16, _z, 0)
    pltpu.sync_copy(zbuf, deg_sh.at[pl.ds(s * RPT, RPT)])
    plsc.subcore_barrier()

    pltpu.sync_copy(col3.at[wid], idx2d)
    for j in range(NCHUNK):
        pltpu.sync_copy(ones.at[pl.ds(0, CH)], deg_sh.at[idx2d.at[j]],
                        add=True)
    plsc.subcore_barrier()
    pltpu.sync_copy(deg_sh.at[pl.ds(s * RPT, RPT)],
                    deg_out.at[c, pl.ds(s * RPT, RPT)])


_deg_kernel = functools.partial(
    pl.kernel,
    out_type=jax.ShapeDtypeStruct((2, NP), jnp.float32),
    mesh=_sc_mesh,
    scratch_types=[
        pltpu.VMEM((NCHUNK, CH), jnp.int32),
        pltpu.VMEM((128,), jnp.float32),
        pltpu.VMEM((RPT,), jnp.float32),
        pltpu.VMEM_SHARED((NP,), jnp.float32),
    ],
)(_deg_body)


def _agg_body(row4, colcat, ycat, acc_out, dst2d, src2d, buf0, buf1, acc_sh,
              sem0, sem1):
    c = lax.axis_index("c")
    s = lax.axis_index("s")

    pltpu.sync_copy(ycat.at[pl.ds(c * NP + s * RPT, RPT)],
                    acc_sh.at[pl.ds(s * RPT, RPT)])
    plsc.subcore_barrier()

    sems = (sem0, sem1)
    bufs = (buf0, buf1)
    for p in range(2):
        pltpu.sync_copy(row4.at[s, pl.ds(p * NCHUNK, NCHUNK)], dst2d)
        pltpu.sync_copy(colcat.at[c, s, pl.ds(p * NCHUNK, NCHUNK)], src2d)

        pending = pltpu.async_copy(ycat.at[src2d.at[0]], buf0, sems[0])
        for j in range(NCHUNK):
            nxt = None
            if j + 1 < NCHUNK:
                nxt = pltpu.async_copy(ycat.at[src2d.at[j + 1]],
                                       bufs[(j + 1) % 2], sems[(j + 1) % 2])
            pending.wait()
            pltpu.sync_copy(bufs[j % 2], acc_sh.at[dst2d.at[j]], add=True)
            pending = nxt

    plsc.subcore_barrier()
    pltpu.sync_copy(acc_sh.at[pl.ds(s * RPT, RPT)],
                    acc_out.at[c, pl.ds(s * RPT, RPT), :])


_agg_kernel = functools.partial(
    pl.kernel,
    out_type=jax.ShapeDtypeStruct((2, NP, DH), jnp.float32),
    mesh=_sc_mesh,
    scratch_types=[
        pltpu.VMEM((NCHUNK, CH), jnp.int32),
        pltpu.VMEM((NCHUNK, CH), jnp.int32),
        pltpu.VMEM((CH, DH), jnp.float32),
        pltpu.VMEM((CH, DH), jnp.float32),
        pltpu.VMEM_SHARED((NP, DH), jnp.float32),
        pltpu.SemaphoreType.DMA,
        pltpu.SemaphoreType.DMA,
    ],
)(_agg_body)


def _lin_body(x_ref, w_ref, b_ref, deg_ref, y_ref):
    xl = lax.dot_general(x_ref[...], w_ref[...], (((1,), (1,)), ((), ())),
                         preferred_element_type=jnp.float32)
    xl = xl + b_ref[...]
    dinv = lax.rsqrt(deg_ref[0, :] + deg_ref[1, :] + 1.0)[:, None]
    y = xl * dinv
    y_ref[0] = y[:, :DH]
    y_ref[1] = y[:, DH:]


def _lin_call(x, w, b2, deg_part):
    blk = NP // 8
    return pl.pallas_call(
        _lin_body,
        grid=(8,),
        in_specs=[
            pl.BlockSpec((blk, D), lambda i: (i, 0)),
            pl.BlockSpec((D, D), lambda i: (0, 0)),
            pl.BlockSpec((1, D), lambda i: (0, 0)),
            pl.BlockSpec((2, blk), lambda i: (0, i)),
        ],
        out_specs=pl.BlockSpec((2, blk, DH), lambda i: (0, i, 0)),
        out_shape=jax.ShapeDtypeStruct((2, NP, DH), jnp.float32),
    )(x, w, b2, deg_part)


def _out_body(acc_ref, deg_ref, o_ref):
    dinv = lax.rsqrt(deg_ref[0, :] + deg_ref[1, :] + 1.0)[:, None]
    o_ref[...] = jnp.concatenate([acc_ref[0] * dinv, acc_ref[1] * dinv],
                                 axis=1)


def _out_call(acc, deg_part):
    blk = NP // 8
    return pl.pallas_call(
        _out_body,
        grid=(8,),
        in_specs=[
            pl.BlockSpec((2, blk, DH), lambda i: (0, i, 0)),
            pl.BlockSpec((2, blk), lambda i: (0, i)),
        ],
        out_specs=pl.BlockSpec((blk, D), lambda i: (i, 0)),
        out_shape=jax.ShapeDtypeStruct((N, D), jnp.float32),
    )(acc, deg_part)


def kernel(x, edge_index, W, b):
    row = edge_index[0]
    col = edge_index[1]
    col3 = col.reshape(32, NCHUNK, CH)
    row4 = row.reshape(16, 2 * NCHUNK, CH)
    col4 = col.reshape(16, 2 * NCHUNK, CH)
    colcat = jnp.stack([col4, col4 + NP])

    deg_part = _deg_kernel(col3)
    y = _lin_call(x, W, b.reshape(1, D), deg_part)
    ycat = y.reshape(2 * NP, DH)
    acc = _agg_kernel(row4, colcat, ycat)
    return _out_call(acc, deg_part)


# speedup vs baseline: 25.8579x; 1.0123x over previous
import functools

import jax
import jax.numpy as jnp
from jax import lax
from jax.experimental import pallas as pl
from jax.experimental.pallas import tpu as pltpu
from jax.experimental.pallas import tpu_sc as plsc

N = 10000
E = 160000
D = 256
DH = 128
NP = 10240
CH = 125
NCHUNK = 40
RPT = NP // 16

_sc_mesh = plsc.VectorSubcoreMesh(core_axis_name="c", subcore_axis_name="s")


def _deg_body(col3, deg_out, idx2d, ones, zbuf, deg_sh, sem):
    c = lax.axis_index("c")
    s = lax.axis_index("s")
    wid = s * 2 + c

    for k in range(8):
        ones[pl.ds(k * 16, 16)] = jnp.ones((16,), jnp.float32)

    def _z(i, _):
        zbuf[pl.ds(i * 16, 16)] = jnp.zeros((16,), jnp.float32)
        return 0

    lax.fori_loop(0, RPT // 16, _z, 0)
    pltpu.sync_copy(zbuf, deg_sh.at[pl.ds(s * RPT, RPT)])
    plsc.subcore_barrier()

    pltpu.sync_copy(col3.at[wid], idx2d)
    descs = [
        pltpu.async_copy(ones.at[pl.ds(0, CH)], deg_sh.at[idx2d.at[j]], sem,
                         add=True)
        for j in range(NCHUNK)
    ]
    for d in descs:
        d.wait()
    plsc.subcore_barrier()
    pltpu.sync_copy(deg_sh.at[pl.ds(s * RPT, RPT)],
                    deg_out.at[c, pl.ds(s * RPT, RPT)])


_deg_kernel = functools.partial(
    pl.kernel,
    out_type=jax.ShapeDtypeStruct((2, NP), jnp.float32),
    mesh=_sc_mesh,
    scratch_types=[
        pltpu.VMEM((NCHUNK, CH), jnp.int32),
        pltpu.VMEM((128,), jnp.float32),
        pltpu.VMEM((RPT,), jnp.float32),
        pltpu.VMEM_SHARED((NP,), jnp.float32),
        pltpu.SemaphoreType.DMA,
    ],
)(_deg_body)


def _agg_body(row4, colcat, ycat, acc_out, dst2d, src2d, buf0, buf1, acc_sh,
              sem0, sem1):
    c = lax.axis_index("c")
    s = lax.axis_index("s")

    pltpu.sync_copy(ycat.at[pl.ds(c * NP + s * RPT, RPT)],
                    acc_sh.at[pl.ds(s * RPT, RPT)])
    plsc.subcore_barrier()

    sems = (sem0, sem1)
    bufs = (buf0, buf1)
    for p in range(2):
        pltpu.sync_copy(row4.at[s, pl.ds(p * NCHUNK, NCHUNK)], dst2d)
        pltpu.sync_copy(colcat.at[c, s, pl.ds(p * NCHUNK, NCHUNK)], src2d)

        pending = pltpu.async_copy(ycat.at[src2d.at[0]], buf0, sems[0])
        for j in range(NCHUNK):
            nxt = None
            if j + 1 < NCHUNK:
                nxt = pltpu.async_copy(ycat.at[src2d.at[j + 1]],
                                       bufs[(j + 1) % 2], sems[(j + 1) % 2])
            pending.wait()
            pltpu.sync_copy(bufs[j % 2], acc_sh.at[dst2d.at[j]], add=True)
            pending = nxt

    plsc.subcore_barrier()
    pltpu.sync_copy(acc_sh.at[pl.ds(s * RPT, RPT)],
                    acc_out.at[c, pl.ds(s * RPT, RPT), :])


_agg_kernel = functools.partial(
    pl.kernel,
    out_type=jax.ShapeDtypeStruct((2, NP, DH), jnp.float32),
    mesh=_sc_mesh,
    scratch_types=[
        pltpu.VMEM((NCHUNK, CH), jnp.int32),
        pltpu.VMEM((NCHUNK, CH), jnp.int32),
        pltpu.VMEM((CH, DH), jnp.float32),
        pltpu.VMEM((CH, DH), jnp.float32),
        pltpu.VMEM_SHARED((NP, DH), jnp.float32),
        pltpu.SemaphoreType.DMA,
        pltpu.SemaphoreType.DMA,
    ],
)(_agg_body)


def _lin_body(x_ref, w_ref, b_ref, deg_ref, y_ref):
    xl = lax.dot_general(x_ref[...], w_ref[...], (((1,), (1,)), ((), ())),
                         preferred_element_type=jnp.float32)
    xl = xl + b_ref[...]
    dinv = lax.rsqrt(deg_ref[0, :] + deg_ref[1, :] + 1.0)[:, None]
    y = xl * dinv
    y_ref[0] = y[:, :DH]
    y_ref[1] = y[:, DH:]


def _lin_call(x, w, b2, deg_part):
    blk = NP // 8
    return pl.pallas_call(
        _lin_body,
        grid=(8,),
        in_specs=[
            pl.BlockSpec((blk, D), lambda i: (i, 0)),
            pl.BlockSpec((D, D), lambda i: (0, 0)),
            pl.BlockSpec((1, D), lambda i: (0, 0)),
            pl.BlockSpec((2, blk), lambda i: (0, i)),
        ],
        out_specs=pl.BlockSpec((2, blk, DH), lambda i: (0, i, 0)),
        out_shape=jax.ShapeDtypeStruct((2, NP, DH), jnp.float32),
    )(x, w, b2, deg_part)


def _out_body(acc_ref, deg_ref, o_ref):
    dinv = lax.rsqrt(deg_ref[0, :] + deg_ref[1, :] + 1.0)[:, None]
    o_ref[...] = jnp.concatenate([acc_ref[0] * dinv, acc_ref[1] * dinv],
                                 axis=1)


def _out_call(acc, deg_part):
    blk = NP // 8
    return pl.pallas_call(
        _out_body,
        grid=(8,),
        in_specs=[
            pl.BlockSpec((2, blk, DH), lambda i: (0, i, 0)),
            pl.BlockSpec((2, blk), lambda i: (0, i)),
        ],
        out_specs=pl.BlockSpec((blk, D), lambda i: (i, 0)),
        out_shape=jax.ShapeDtypeStruct((N, D), jnp.float32),
    )(acc, deg_part)


def kernel(x, edge_index, W, b):
    row = edge_index[0]
    col = edge_index[1]
    col3 = col.reshape(32, NCHUNK, CH)
    row4 = row.reshape(16, 2 * NCHUNK, CH)
    col4 = col.reshape(16, 2 * NCHUNK, CH)
    colcat = jnp.stack([col4, col4 + NP])

    deg_part = _deg_kernel(col3)
    y = _lin_call(x, W, b.reshape(1, D), deg_part)
    ycat = y.reshape(2 * NP, DH)
    acc = _agg_kernel(row4, colcat, ycat)
    return _out_call(acc, deg_part)
